# Initial kernel scaffold; baseline (speedup 1.0000x reference)
import jax
import jax.numpy as jnp
from jax.experimental import pallas as pl


def kernel(x, W_gate):
    raise NotImplementedError("write your pallas kernel here")


# speedup vs baseline: 1.7640x; 1.7640x over previous
import functools

import jax
import jax.numpy as jnp
from jax.experimental import pallas as pl
from jax.experimental.pallas import tpu as pltpu

D_MODEL_K = 768
N_EXP = 64
BLK = 1024


def _router_body(x_ref, w_ref, wts_ref, idx_ref, logits_ref):
    x = x_ref[...]
    w = w_ref[...]
    logits = jax.lax.dot_general(
        x, w, (((1,), (1,)), ((), ())), preferred_element_type=jnp.float32
    )
    logits_ref[...] = logits

    m = jnp.max(logits, axis=-1, keepdims=True)
    e = jnp.exp(logits - m)
    p = e / jnp.sum(e, axis=-1, keepdims=True)

    lane = jax.lax.broadcasted_iota(jnp.int32, p.shape, 1)
    m1 = jnp.max(p, axis=-1, keepdims=True)
    i1 = jnp.min(jnp.where(p == m1, lane, N_EXP), axis=-1, keepdims=True)
    p_masked = jnp.where(lane == i1, -jnp.inf, p)
    m2 = jnp.max(p_masked, axis=-1, keepdims=True)
    i2 = jnp.min(jnp.where(p_masked == m2, lane, N_EXP), axis=-1, keepdims=True)

    denom = m1 + m2 + 1e-10
    wts_ref[...] = jnp.concatenate([m1 / denom, m2 / denom], axis=-1)
    idx_ref[...] = jnp.concatenate([i1, i2], axis=-1)


@jax.jit
def kernel(x, W_gate):
    batch, seq_len, d_model = x.shape
    n_rows = batch * seq_len
    x_flat = x.reshape(n_rows, d_model)
    grid = (n_rows // BLK,)
    wts, idx, logits = pl.pallas_call(
        _router_body,
        grid=grid,
        in_specs=[
            pl.BlockSpec((BLK, d_model), lambda i: (i, 0)),
            pl.BlockSpec((N_EXP, d_model), lambda i: (0, 0)),
        ],
        out_specs=[
            pl.BlockSpec((BLK, 2), lambda i: (i, 0)),
            pl.BlockSpec((BLK, 2), lambda i: (i, 0)),
            pl.BlockSpec((BLK, N_EXP), lambda i: (i, 0)),
        ],
        out_shape=[
            jax.ShapeDtypeStruct((n_rows, 2), jnp.float32),
            jax.ShapeDtypeStruct((n_rows, 2), jnp.int32),
            jax.ShapeDtypeStruct((n_rows, N_EXP), jnp.float32),
        ],
        compiler_params=pltpu.CompilerParams(
            dimension_semantics=("arbitrary",),
        ),
    )(x_flat, W_gate)
    return (wts, idx, logits)


# speedup vs baseline: 1.9961x; 1.1316x over previous
import functools

import jax
import jax.numpy as jnp
from jax.experimental import pallas as pl
from jax.experimental.pallas import tpu as pltpu

D_MODEL_K = 768
N_EXP = 64
BLK = 2048


def _router_body(x_ref, w_ref, wts_ref, idx_ref, logits_ref):
    x = x_ref[...]
    w = w_ref[...]
    logits = jax.lax.dot_general(
        x, w, (((1,), (1,)), ((), ())), preferred_element_type=jnp.float32
    )
    logits_ref[...] = logits

    m = jnp.max(logits, axis=-1, keepdims=True)
    e = jnp.exp(logits - m)
    p = e / jnp.sum(e, axis=-1, keepdims=True)

    lane = jax.lax.broadcasted_iota(jnp.int32, p.shape, 1)
    m1 = jnp.max(p, axis=-1, keepdims=True)
    i1 = jnp.min(jnp.where(p == m1, lane, N_EXP), axis=-1, keepdims=True)
    p_masked = jnp.where(lane == i1, -jnp.inf, p)
    m2 = jnp.max(p_masked, axis=-1, keepdims=True)
    i2 = jnp.min(jnp.where(p_masked == m2, lane, N_EXP), axis=-1, keepdims=True)

    denom = m1 + m2 + 1e-10
    wts_ref[...] = jnp.concatenate([m1 / denom, m2 / denom], axis=-1)
    idx_ref[...] = jnp.concatenate([i1, i2], axis=-1)


@jax.jit
def kernel(x, W_gate):
    batch, seq_len, d_model = x.shape
    n_rows = batch * seq_len
    x_flat = x.reshape(n_rows, d_model)
    grid = (n_rows // BLK,)
    wts, idx, logits = pl.pallas_call(
        _router_body,
        grid=grid,
        in_specs=[
            pl.BlockSpec((BLK, d_model), lambda i: (i, 0)),
            pl.BlockSpec((N_EXP, d_model), lambda i: (0, 0)),
        ],
        out_specs=[
            pl.BlockSpec((BLK, 2), lambda i: (i, 0)),
            pl.BlockSpec((BLK, 2), lambda i: (i, 0)),
            pl.BlockSpec((BLK, N_EXP), lambda i: (i, 0)),
        ],
        out_shape=[
            jax.ShapeDtypeStruct((n_rows, 2), jnp.float32),
            jax.ShapeDtypeStruct((n_rows, 2), jnp.int32),
            jax.ShapeDtypeStruct((n_rows, N_EXP), jnp.float32),
        ],
        compiler_params=pltpu.CompilerParams(
            dimension_semantics=("arbitrary",),
        ),
    )(x_flat, W_gate)
    return (wts, idx, logits)


# speedup vs baseline: 2.0977x; 1.0509x over previous
import functools

import jax
import jax.numpy as jnp
from jax.experimental import pallas as pl
from jax.experimental.pallas import tpu as pltpu

D_MODEL_K = 768
N_EXP = 64
BLK = 4096


def _router_body(x_ref, w_ref, wts_ref, idx_ref, logits_ref):
    x = x_ref[...]
    w = w_ref[...]
    logits = jax.lax.dot_general(
        x, w, (((1,), (1,)), ((), ())), preferred_element_type=jnp.float32
    )
    logits_ref[...] = logits

    m = jnp.max(logits, axis=-1, keepdims=True)
    e = jnp.exp(logits - m)
    p = e / jnp.sum(e, axis=-1, keepdims=True)

    lane = jax.lax.broadcasted_iota(jnp.int32, p.shape, 1)
    m1 = jnp.max(p, axis=-1, keepdims=True)
    i1 = jnp.min(jnp.where(p == m1, lane, N_EXP), axis=-1, keepdims=True)
    p_masked = jnp.where(lane == i1, -jnp.inf, p)
    m2 = jnp.max(p_masked, axis=-1, keepdims=True)
    i2 = jnp.min(jnp.where(p_masked == m2, lane, N_EXP), axis=-1, keepdims=True)

    denom = m1 + m2 + 1e-10
    wts_ref[...] = jnp.concatenate([m1 / denom, m2 / denom], axis=-1)
    idx_ref[...] = jnp.concatenate([i1, i2], axis=-1)


@jax.jit
def kernel(x, W_gate):
    batch, seq_len, d_model = x.shape
    n_rows = batch * seq_len
    x_flat = x.reshape(n_rows, d_model)
    grid = (n_rows // BLK,)
    wts, idx, logits = pl.pallas_call(
        _router_body,
        grid=grid,
        in_specs=[
            pl.BlockSpec((BLK, d_model), lambda i: (i, 0)),
            pl.BlockSpec((N_EXP, d_model), lambda i: (0, 0)),
        ],
        out_specs=[
            pl.BlockSpec((BLK, 2), lambda i: (i, 0)),
            pl.BlockSpec((BLK, 2), lambda i: (i, 0)),
            pl.BlockSpec((BLK, N_EXP), lambda i: (i, 0)),
        ],
        out_shape=[
            jax.ShapeDtypeStruct((n_rows, 2), jnp.float32),
            jax.ShapeDtypeStruct((n_rows, 2), jnp.int32),
            jax.ShapeDtypeStruct((n_rows, N_EXP), jnp.float32),
        ],
        compiler_params=pltpu.CompilerParams(
            dimension_semantics=("arbitrary",),
        ),
    )(x_flat, W_gate)
    return (wts, idx, logits)


# speedup vs baseline: 2.1404x; 1.0204x over previous
import functools

import jax
import jax.numpy as jnp
from jax.experimental import pallas as pl
from jax.experimental.pallas import tpu as pltpu

D_MODEL_K = 768
N_EXP = 64
BLK = 4096


def _router_body(x_ref, w_ref, wts_ref, idx_ref, logits_ref):
    x = x_ref[...]
    w = w_ref[...]
    logits = jax.lax.dot_general(
        x, w, (((1,), (1,)), ((), ())), preferred_element_type=jnp.float32
    )
    logits_ref[...] = logits

    lane = jax.lax.broadcasted_iota(jnp.int32, logits.shape, 1)
    m1 = jnp.max(logits, axis=-1, keepdims=True)
    i1 = jnp.min(jnp.where(logits == m1, lane, N_EXP), axis=-1, keepdims=True)
    l_masked = jnp.where(lane == i1, -jnp.inf, logits)
    m2 = jnp.max(l_masked, axis=-1, keepdims=True)
    i2 = jnp.min(jnp.where(l_masked == m2, lane, N_EXP), axis=-1, keepdims=True)

    w1 = 1.0 / (1.0 + jnp.exp(m2 - m1))
    wts_ref[...] = jnp.concatenate([w1, 1.0 - w1], axis=-1)
    idx_ref[...] = jnp.concatenate([i1, i2], axis=-1)


@jax.jit
def kernel(x, W_gate):
    batch, seq_len, d_model = x.shape
    n_rows = batch * seq_len
    x_flat = x.reshape(n_rows, d_model)
    grid = (n_rows // BLK,)
    wts, idx, logits = pl.pallas_call(
        _router_body,
        grid=grid,
        in_specs=[
            pl.BlockSpec((BLK, d_model), lambda i: (i, 0)),
            pl.BlockSpec((N_EXP, d_model), lambda i: (0, 0)),
        ],
        out_specs=[
            pl.BlockSpec((BLK, 2), lambda i: (i, 0)),
            pl.BlockSpec((BLK, 2), lambda i: (i, 0)),
            pl.BlockSpec((BLK, N_EXP), lambda i: (i, 0)),
        ],
        out_shape=[
            jax.ShapeDtypeStruct((n_rows, 2), jnp.float32),
            jax.ShapeDtypeStruct((n_rows, 2), jnp.int32),
            jax.ShapeDtypeStruct((n_rows, N_EXP), jnp.float32),
        ],
        compiler_params=pltpu.CompilerParams(
            dimension_semantics=("arbitrary",),
        ),
    )(x_flat, W_gate)
    return (wts, idx, logits)
